# baseline (speedup 1.0000x reference)
import jax
import jax.numpy as jnp
from jax.experimental import pallas as pl
from jax.experimental.pallas import tpu as pltpu

_N = 10000
_D = 128
_BR = 400
_STEPS = _N // _BR


def _gcn_body(a_ref, x_ref, w_ref, b_ref, al_ref, o_ref, sup_s):
    i = pl.program_id(0)

    @pl.when(i == 0)
    def _prep():
        xw = jnp.dot(x_ref[...].astype(jnp.bfloat16),
                     w_ref[...].astype(jnp.bfloat16),
                     preferred_element_type=jnp.float32)
        sup_s[...] = xw.astype(jnp.bfloat16)

    asup = jnp.dot(a_ref[...].astype(jnp.bfloat16), sup_s[...],
                   preferred_element_type=jnp.float32)
    xblk = x_ref[pl.ds(i * _BR, _BR), :]
    l0 = al_ref[0, 0]
    l1 = al_ref[0, 1]
    m = jnp.maximum(l0, l1)
    e0 = jnp.exp(l0 - m)
    e1 = jnp.exp(l1 - m)
    g0 = e0 / (e0 + e1)
    g1 = e1 / (e0 + e1)
    y = g0 * asup + g1 * xblk + b_ref[...]
    o_ref[...] = jnp.where(y > 0.0, y, jnp.exp(jnp.minimum(y, 0.0)) - 1.0)


def kernel(inputs, adj, weight, bias, alpha):
    bias2 = bias.reshape(1, _D)
    al2 = alpha.reshape(1, 2)
    return pl.pallas_call(
        _gcn_body,
        grid=(_STEPS,),
        in_specs=[
            pl.BlockSpec((_BR, _N), lambda i: (i, 0)),
            pl.BlockSpec((_N, _D), lambda i: (0, 0)),
            pl.BlockSpec((_D, _D), lambda i: (0, 0)),
            pl.BlockSpec((1, _D), lambda i: (0, 0)),
            pl.BlockSpec((1, 2), lambda i: (0, 0)),
        ],
        out_specs=pl.BlockSpec((_BR, _D), lambda i: (i, 0)),
        out_shape=jax.ShapeDtypeStruct((_N, _D), jnp.float32),
        scratch_shapes=[
            pltpu.VMEM((_N, _D), jnp.bfloat16),
        ],
        compiler_params=pltpu.CompilerParams(
            dimension_semantics=("arbitrary",),
        ),
    )(adj, inputs, weight, bias2, al2)
